# Initial kernel scaffold; baseline (speedup 1.0000x reference)
import jax
import jax.numpy as jnp
from jax.experimental import pallas as pl


def kernel(x, edge_index, W_neigh0, b0, W_self0, W_neigh1, b1, W_self1, W_neigh2, b2, W_self2):
    raise NotImplementedError("write your pallas kernel here")


# speedup vs baseline: 52.6351x; 52.6351x over previous
import functools

import jax
import jax.numpy as jnp
from jax import lax
from jax.experimental import pallas as pl
from jax.experimental.pallas import tpu as pltpu
from jax.experimental.pallas import tpu_sc as plsc

N = 10000
D = 128
E = 320000
NP = 10240
NT = 16
NPT = NP // NT
EPT = 20096
EPAD = NT * EPT - E


def _tc_body(x_ref, w2_ref, ms_ref, cs_ref):
    xb = x_ref[...]
    ms_ref[...] = jnp.dot(xb, w2_ref[...], preferred_element_type=jnp.float32)
    cs_ref[...] = jnp.sum(xb, axis=0)


def _sc_body(m0_hbm, s0_hbm, src_hbm, dst_hbm, scal_hbm, out_hbm,
             partials_sp, v_sp, src_v, dst_v, v_loc, agg_loc, pw_in,
             hnew, s0_loc, scal_loc, sums_loc):
    w = lax.axis_index("s")
    nbase = w * NPT

    pltpu.sync_copy(src_hbm.at[pl.ds(w * EPT, EPT)], src_v)
    pltpu.sync_copy(dst_hbm.at[pl.ds(w * EPT, EPT)], dst_v)
    pltpu.sync_copy(scal_hbm, scal_loc)
    pltpu.sync_copy(s0_hbm.at[pl.ds(nbase, NPT)], s0_loc)
    pltpu.sync_copy(m0_hbm, v_loc)

    zero16 = jnp.zeros((16,), jnp.float32)
    iota16 = lax.iota(jnp.int32, 16)

    for l in range(3):
        def zbody(i, c):
            agg_loc[pl.ds(i * 16, 16)] = zero16
            return c
        lax.fori_loop(0, NP // 16, zbody, 0)

        def ebody(i, c):
            base = i * 64
            for u in range(4):
                off = base + u * 16
                sidx = src_v[pl.ds(off, 16)]
                didx = dst_v[pl.ds(off, 16)]
                vals = plsc.load_gather(v_loc, [sidx])
                plsc.addupdate_scatter(agg_loc, [didx], vals)
            return c
        lax.fori_loop(0, EPT // 64, ebody, 0)

        pltpu.sync_copy(agg_loc, partials_sp.at[w])
        plsc.subcore_barrier()
        pltpu.sync_copy(partials_sp.at[:, pl.ds(nbase, NPT)], pw_in)

        scv = scal_loc[...]
        if l == 0:
            wb = scv[0]
        elif l == 1:
            wn, wb, ws = scv[1], scv[2], scv[3]
        else:
            wn, wb, ws = scv[4], scv[5], scv[6]

        def pbody(j, sacc):
            off = j * 16
            acc = pw_in[0, pl.ds(off, 16)]
            for t in range(1, NT):
                acc = acc + pw_in[t, pl.ds(off, 16)]
            if l == 0:
                h = acc + wb + s0_loc[pl.ds(off, 16)]
            else:
                h = wn * acc + wb + ws * v_loc[pl.ds(nbase + off, 16)]
            h = jnp.maximum(h, 0.0)
            nid = nbase + off + iota16
            h = jnp.where(nid < N, h, 0.0)
            hnew[pl.ds(off, 16)] = h
            return sacc + h
        sacc = lax.fori_loop(0, NPT // 16, pbody, zero16)
        sums_loc[...] = sacc
        pltpu.sync_copy(hnew, v_sp.at[pl.ds(nbase, NPT)])
        pltpu.sync_copy(sums_loc, out_hbm.at[l, w])
        plsc.subcore_barrier()
        if l < 2:
            pltpu.sync_copy(v_sp, v_loc)


@functools.partial(
    pl.kernel,
    out_type=jax.ShapeDtypeStruct((3, NT, 16), jnp.float32),
    mesh=plsc.VectorSubcoreMesh(
        core_axis_name="c", subcore_axis_name="s",
        num_cores=1, num_subcores=NT),
    scratch_types=[
        pltpu.VMEM_SHARED((NT, NP), jnp.float32),
        pltpu.VMEM_SHARED((NP,), jnp.float32),
        pltpu.VMEM((EPT,), jnp.int32),
        pltpu.VMEM((EPT,), jnp.int32),
        pltpu.VMEM((NP,), jnp.float32),
        pltpu.VMEM((NP,), jnp.float32),
        pltpu.VMEM((NT, NPT), jnp.float32),
        pltpu.VMEM((NPT,), jnp.float32),
        pltpu.VMEM((NPT,), jnp.float32),
        pltpu.VMEM((16,), jnp.float32),
        pltpu.VMEM((16,), jnp.float32),
    ],
    compiler_params=pltpu.CompilerParams(needs_layout_passes=False),
)
def _sc_edges(m0_hbm, s0_hbm, src_hbm, dst_hbm, scal_hbm, out_hbm,
              partials_sp, v_sp, src_v, dst_v, v_loc, agg_loc, pw_in,
              hnew, s0_loc, scal_loc, sums_loc):
    _sc_body(m0_hbm, s0_hbm, src_hbm, dst_hbm, scal_hbm, out_hbm,
             partials_sp, v_sp, src_v, dst_v, v_loc, agg_loc, pw_in,
             hnew, s0_loc, scal_loc, sums_loc)


def kernel(x, edge_index, W_neigh0, b0, W_self0, W_neigh1, b1, W_self1,
           W_neigh2, b2, W_self2):
    ms, colsum = pl.pallas_call(
        _tc_body,
        out_shape=[
            jax.ShapeDtypeStruct((N, 2), jnp.float32),
            jax.ShapeDtypeStruct((D,), jnp.float32),
        ],
    )(x, jnp.concatenate([W_neigh0, W_self0], axis=1))

    zpad = jnp.zeros((NP - N,), jnp.float32)
    m0p = jnp.concatenate([ms[:, 0], zpad])
    s0p = jnp.concatenate([ms[:, 1], zpad])
    ipad = jnp.full((EPAD,), NP - 1, jnp.int32)
    srcp = jnp.concatenate([edge_index[0], ipad])
    dstp = jnp.concatenate([edge_index[1], ipad])
    scal = jnp.concatenate([
        b0, W_neigh1[0], b1, W_self1[0], W_neigh2[0], b2, W_self2[0],
        jnp.zeros((9,), jnp.float32),
    ])

    sums = _sc_edges(m0p, s0p, srcp, dstp, scal)
    hsums = jnp.sum(sums, axis=(1, 2))
    return jnp.concatenate([colsum, hsums])


# speedup vs baseline: 63.6703x; 1.2097x over previous
import functools

import jax
import jax.numpy as jnp
from jax import lax
from jax.experimental import pallas as pl
from jax.experimental.pallas import tpu as pltpu
from jax.experimental.pallas import tpu_sc as plsc

N = 10000
D = 128
E = 320000
NP = 10240
NT = 16
NPT = NP // NT
EPT = E // NT
EUNROLL = 10


def _tc_body(x_ref, wn_ref, ws_ref, m_ref, s_ref, cs_ref):
    xb = x_ref[...]
    zt = jnp.zeros((NP - N, 1), jnp.float32)
    m_ref[pl.ds(0, N), :] = jnp.dot(xb, wn_ref[...],
                                    preferred_element_type=jnp.float32)
    m_ref[pl.ds(N, NP - N), :] = zt
    s_ref[pl.ds(0, N), :] = jnp.dot(xb, ws_ref[...],
                                    preferred_element_type=jnp.float32)
    s_ref[pl.ds(N, NP - N), :] = zt
    cs_ref[...] = jnp.sum(xb, axis=0)


def _sc_body(m0_hbm, s0_hbm, ei_hbm, scal_hbm, out_hbm,
             partials_sp, v_sp, src_v, dst_v, v_loc, agg_loc, pw_in,
             hnew, s0_loc, scal_loc, sums_loc):
    w = lax.axis_index("s")
    nbase = w * NPT

    pltpu.sync_copy(ei_hbm.at[pl.ds(w * EPT, EPT)], src_v)
    pltpu.sync_copy(ei_hbm.at[pl.ds(E + w * EPT, EPT)], dst_v)
    pltpu.sync_copy(scal_hbm, scal_loc)
    pltpu.sync_copy(s0_hbm.at[pl.ds(nbase, NPT)], s0_loc)
    pltpu.sync_copy(m0_hbm, v_loc)

    zero16 = jnp.zeros((16,), jnp.float32)
    iota16 = lax.iota(jnp.int32, 16)

    for l in range(3):
        def zbody(i, c):
            base = i * 128
            for u in range(8):
                agg_loc[pl.ds(base + u * 16, 16)] = zero16
            return c
        lax.fori_loop(0, NP // 128, zbody, 0)

        def ebody(i, c):
            base = i * (16 * EUNROLL)
            for u in range(EUNROLL):
                off = base + u * 16
                sidx = src_v[pl.ds(off, 16)]
                didx = dst_v[pl.ds(off, 16)]
                vals = plsc.load_gather(v_loc, [sidx])
                plsc.addupdate_scatter(agg_loc, [didx], vals)
            return c
        lax.fori_loop(0, EPT // (16 * EUNROLL), ebody, 0)

        pltpu.sync_copy(agg_loc, partials_sp.at[w])
        plsc.subcore_barrier()
        pltpu.sync_copy(partials_sp.at[:, pl.ds(nbase, NPT)], pw_in)

        scv = scal_loc[...]
        if l == 0:
            wb = scv[0]
        elif l == 1:
            wn, wb, ws = scv[1], scv[2], scv[3]
        else:
            wn, wb, ws = scv[4], scv[5], scv[6]

        def pbody(j, sacc):
            for u in range(4):
                off = (j * 4 + u) * 16
                acc = pw_in[0, pl.ds(off, 16)]
                for t in range(1, NT):
                    acc = acc + pw_in[t, pl.ds(off, 16)]
                if l == 0:
                    h = acc + wb + s0_loc[pl.ds(off, 16)]
                else:
                    h = wn * acc + wb + ws * v_loc[pl.ds(nbase + off, 16)]
                h = jnp.maximum(h, 0.0)
                nid = nbase + off + iota16
                h = jnp.where(nid < N, h, 0.0)
                hnew[pl.ds(off, 16)] = h
                sacc = sacc + h
            return sacc
        sacc = lax.fori_loop(0, NPT // 64, pbody, zero16)
        sums_loc[...] = sacc
        pltpu.sync_copy(hnew, v_sp.at[pl.ds(nbase, NPT)])
        pltpu.sync_copy(sums_loc, out_hbm.at[l, w])
        plsc.subcore_barrier()
        if l < 2:
            pltpu.sync_copy(v_sp, v_loc)


@functools.partial(
    pl.kernel,
    out_type=jax.ShapeDtypeStruct((3, NT, 16), jnp.float32),
    mesh=plsc.VectorSubcoreMesh(
        core_axis_name="c", subcore_axis_name="s",
        num_cores=1, num_subcores=NT),
    scratch_types=[
        pltpu.VMEM_SHARED((NT, NP), jnp.float32),
        pltpu.VMEM_SHARED((NP,), jnp.float32),
        pltpu.VMEM((EPT,), jnp.int32),
        pltpu.VMEM((EPT,), jnp.int32),
        pltpu.VMEM((NP,), jnp.float32),
        pltpu.VMEM((NP,), jnp.float32),
        pltpu.VMEM((NT, NPT), jnp.float32),
        pltpu.VMEM((NPT,), jnp.float32),
        pltpu.VMEM((NPT,), jnp.float32),
        pltpu.VMEM((16,), jnp.float32),
        pltpu.VMEM((16,), jnp.float32),
    ],
    compiler_params=pltpu.CompilerParams(needs_layout_passes=False),
)
def _sc_edges(m0_hbm, s0_hbm, ei_hbm, scal_hbm, out_hbm,
              partials_sp, v_sp, src_v, dst_v, v_loc, agg_loc, pw_in,
              hnew, s0_loc, scal_loc, sums_loc):
    _sc_body(m0_hbm, s0_hbm, ei_hbm, scal_hbm, out_hbm,
             partials_sp, v_sp, src_v, dst_v, v_loc, agg_loc, pw_in,
             hnew, s0_loc, scal_loc, sums_loc)


def kernel(x, edge_index, W_neigh0, b0, W_self0, W_neigh1, b1, W_self1,
           W_neigh2, b2, W_self2):
    m0, s0, colsum = pl.pallas_call(
        _tc_body,
        out_shape=[
            jax.ShapeDtypeStruct((NP, 1), jnp.float32),
            jax.ShapeDtypeStruct((NP, 1), jnp.float32),
            jax.ShapeDtypeStruct((D,), jnp.float32),
        ],
    )(x, W_neigh0, W_self0)

    scal = jnp.concatenate([
        b0, W_neigh1[0], b1, W_self1[0], W_neigh2[0], b2, W_self2[0],
        jnp.zeros((9,), jnp.float32),
    ])

    sums = _sc_edges(m0.reshape(NP), s0.reshape(NP),
                     edge_index.reshape(2 * E), scal)
    hsums = jnp.sum(sums, axis=(1, 2))
    return jnp.concatenate([colsum, hsums])


# speedup vs baseline: 84.4962x; 1.3271x over previous
import functools

import jax
import jax.numpy as jnp
from jax import lax
from jax.experimental import pallas as pl
from jax.experimental.pallas import tpu as pltpu
from jax.experimental.pallas import tpu_sc as plsc

N = 10000
D = 128
E = 320000
NP = 10240
NT = 16
NPT = NP // NT
EPT = E // NT
EUNROLL = 10


def _tc_body(x_ref, wn_ref, ws_ref, m_ref, s_ref, cs_ref):
    xb = x_ref[...]
    zt = jnp.zeros((NP - N, 1), jnp.float32)
    m_ref[pl.ds(0, N), :] = jnp.dot(xb, wn_ref[...],
                                    preferred_element_type=jnp.float32)
    m_ref[pl.ds(N, NP - N), :] = zt
    s_ref[pl.ds(0, N), :] = jnp.dot(xb, ws_ref[...],
                                    preferred_element_type=jnp.float32)
    s_ref[pl.ds(N, NP - N), :] = zt
    cs_ref[...] = jnp.sum(xb, axis=0)


def _sc_body(m0_hbm, s0_hbm, ei_hbm, scal_hbm, out_hbm,
             partials_sp, v_sp, src_v, dst_v, v_loc, agg_loc, pw_in,
             hnew, s0_loc, scal_loc, sums_loc):
    w = lax.axis_index("s")
    nbase = w * NPT

    pltpu.sync_copy(ei_hbm.at[pl.ds(w * EPT, EPT)], src_v)
    pltpu.sync_copy(ei_hbm.at[pl.ds(E + w * EPT, EPT)], dst_v)
    pltpu.sync_copy(scal_hbm, scal_loc)
    pltpu.sync_copy(s0_hbm.at[pl.ds(nbase, NPT)], s0_loc)
    pltpu.sync_copy(m0_hbm, v_loc)

    zero16 = jnp.zeros((16,), jnp.float32)
    iota16 = lax.iota(jnp.int32, 16)

    for l in range(3):
        def zbody(i, c):
            base = i * 128
            for u in range(8):
                agg_loc[pl.ds(base + u * 16, 16)] = zero16
            return c
        lax.fori_loop(0, NP // 128, zbody, 0)

        @plsc.parallel_loop(0, EPT // 16, unroll=EUNROLL)
        def _(i):
            off = i * 16
            sidx = src_v[pl.ds(off, 16)]
            didx = dst_v[pl.ds(off, 16)]
            vals = plsc.load_gather(v_loc, [sidx])
            plsc.addupdate_scatter(agg_loc, [didx], vals)

        pltpu.sync_copy(agg_loc, partials_sp.at[w])
        plsc.subcore_barrier()
        pltpu.sync_copy(partials_sp.at[:, pl.ds(nbase, NPT)], pw_in)

        scv = scal_loc[...]
        if l == 0:
            wb = scv[0]
        elif l == 1:
            wn, wb, ws = scv[1], scv[2], scv[3]
        else:
            wn, wb, ws = scv[4], scv[5], scv[6]

        def pbody(j, sacc):
            for u in range(4):
                off = (j * 4 + u) * 16
                acc = pw_in[0, pl.ds(off, 16)]
                for t in range(1, NT):
                    acc = acc + pw_in[t, pl.ds(off, 16)]
                if l == 0:
                    h = acc + wb + s0_loc[pl.ds(off, 16)]
                else:
                    h = wn * acc + wb + ws * v_loc[pl.ds(nbase + off, 16)]
                h = jnp.maximum(h, 0.0)
                nid = nbase + off + iota16
                h = jnp.where(nid < N, h, 0.0)
                hnew[pl.ds(off, 16)] = h
                sacc = sacc + h
            return sacc
        sacc = lax.fori_loop(0, NPT // 64, pbody, zero16)
        sums_loc[...] = sacc
        pltpu.sync_copy(hnew, v_sp.at[pl.ds(nbase, NPT)])
        pltpu.sync_copy(sums_loc, out_hbm.at[l, w])
        plsc.subcore_barrier()
        if l < 2:
            pltpu.sync_copy(v_sp, v_loc)


@functools.partial(
    pl.kernel,
    out_type=jax.ShapeDtypeStruct((3, NT, 16), jnp.float32),
    mesh=plsc.VectorSubcoreMesh(
        core_axis_name="c", subcore_axis_name="s",
        num_cores=1, num_subcores=NT),
    scratch_types=[
        pltpu.VMEM_SHARED((NT, NP), jnp.float32),
        pltpu.VMEM_SHARED((NP,), jnp.float32),
        pltpu.VMEM((EPT,), jnp.int32),
        pltpu.VMEM((EPT,), jnp.int32),
        pltpu.VMEM((NP,), jnp.float32),
        pltpu.VMEM((NP,), jnp.float32),
        pltpu.VMEM((NT, NPT), jnp.float32),
        pltpu.VMEM((NPT,), jnp.float32),
        pltpu.VMEM((NPT,), jnp.float32),
        pltpu.VMEM((16,), jnp.float32),
        pltpu.VMEM((16,), jnp.float32),
    ],
    compiler_params=pltpu.CompilerParams(needs_layout_passes=False),
)
def _sc_edges(m0_hbm, s0_hbm, ei_hbm, scal_hbm, out_hbm,
              partials_sp, v_sp, src_v, dst_v, v_loc, agg_loc, pw_in,
              hnew, s0_loc, scal_loc, sums_loc):
    _sc_body(m0_hbm, s0_hbm, ei_hbm, scal_hbm, out_hbm,
             partials_sp, v_sp, src_v, dst_v, v_loc, agg_loc, pw_in,
             hnew, s0_loc, scal_loc, sums_loc)


def kernel(x, edge_index, W_neigh0, b0, W_self0, W_neigh1, b1, W_self1,
           W_neigh2, b2, W_self2):
    m0, s0, colsum = pl.pallas_call(
        _tc_body,
        out_shape=[
            jax.ShapeDtypeStruct((NP, 1), jnp.float32),
            jax.ShapeDtypeStruct((NP, 1), jnp.float32),
            jax.ShapeDtypeStruct((D,), jnp.float32),
        ],
    )(x, W_neigh0, W_self0)

    scal = jnp.concatenate([
        b0, W_neigh1[0], b1, W_self1[0], W_neigh2[0], b2, W_self2[0],
        jnp.zeros((9,), jnp.float32),
    ])

    sums = _sc_edges(m0.reshape(NP), s0.reshape(NP),
                     edge_index.reshape(2 * E), scal)
    hsums = jnp.sum(sums, axis=(1, 2))
    return jnp.concatenate([colsum, hsums])


# speedup vs baseline: 89.6014x; 1.0604x over previous
import functools

import jax
import jax.numpy as jnp
from jax import lax
from jax.experimental import pallas as pl
from jax.experimental.pallas import tpu as pltpu
from jax.experimental.pallas import tpu_sc as plsc

N = 10000
D = 128
E = 320000
NP = 10240
NT = 16
NPT = NP // NT
EPT = E // NT
EUNROLL = 10


def _tc_body(x_ref, wn_ref, ws_ref, m_ref, s_ref, cs_ref):
    xb = x_ref[...]
    zt = jnp.zeros((NP - N, 1), jnp.float32)
    m_ref[pl.ds(0, N), :] = jnp.dot(xb, wn_ref[...],
                                    preferred_element_type=jnp.float32)
    m_ref[pl.ds(N, NP - N), :] = zt
    s_ref[pl.ds(0, N), :] = jnp.dot(xb, ws_ref[...],
                                    preferred_element_type=jnp.float32)
    s_ref[pl.ds(N, NP - N), :] = zt
    cs_ref[...] = jnp.sum(xb, axis=0)


def _sc_body(m0_hbm, s0_hbm, ei_hbm, scal_hbm, out_hbm,
             partials_sp, v_sp, src_v, dst_v, v_loc, agg_loc, pw_in,
             hnew, s0_loc, scal_loc, sums_loc, dma_sem):
    w = lax.axis_index("s")
    nbase = w * NPT

    cp_src = pltpu.async_copy(ei_hbm.at[pl.ds(w * EPT, EPT)], src_v, dma_sem)
    cp_dst = pltpu.async_copy(ei_hbm.at[pl.ds(E + w * EPT, EPT)], dst_v,
                              dma_sem)
    pltpu.sync_copy(scal_hbm, scal_loc)
    pltpu.sync_copy(s0_hbm.at[pl.ds(nbase, NPT)], s0_loc)
    pltpu.sync_copy(m0_hbm, v_loc)
    cp_src.wait()
    cp_dst.wait()

    zero16 = jnp.zeros((16,), jnp.float32)
    iota16 = lax.iota(jnp.int32, 16)

    for l in range(3):
        @plsc.parallel_loop(0, NP // 128, unroll=4)
        def _(i):
            base = i * 128
            for u in range(8):
                agg_loc[pl.ds(base + u * 16, 16)] = zero16

        @plsc.parallel_loop(0, EPT // 16, unroll=EUNROLL)
        def _(i):
            off = i * 16
            sidx = src_v[pl.ds(off, 16)]
            didx = dst_v[pl.ds(off, 16)]
            vals = plsc.load_gather(v_loc, [sidx])
            plsc.addupdate_scatter(agg_loc, [didx], vals)

        cps = [pltpu.async_copy(agg_loc.at[pl.ds(t * NPT, NPT)],
                                partials_sp.at[t, w], dma_sem)
               for t in range(NT)]
        for cp in cps:
            cp.wait()
        plsc.subcore_barrier()
        pltpu.sync_copy(partials_sp.at[w], pw_in)

        scv = scal_loc[...]
        if l == 0:
            wb = scv[0]
        elif l == 1:
            wn, wb, ws = scv[1], scv[2], scv[3]
        else:
            wn, wb, ws = scv[4], scv[5], scv[6]

        @plsc.parallel_loop(0, NPT // 16, unroll=4, carry=zero16)
        def sacc(j, sc):
            off = j * 16
            acc = pw_in[0, pl.ds(off, 16)]
            for t in range(1, NT):
                acc = acc + pw_in[t, pl.ds(off, 16)]
            if l == 0:
                h = acc + wb + s0_loc[pl.ds(off, 16)]
            else:
                h = wn * acc + wb + ws * v_loc[pl.ds(nbase + off, 16)]
            h = jnp.maximum(h, 0.0)
            nid = nbase + off + iota16
            h = jnp.where(nid < N, h, 0.0)
            hnew[pl.ds(off, 16)] = h
            return sc + h
        sums_loc[...] = sacc
        pltpu.sync_copy(hnew, v_sp.at[pl.ds(nbase, NPT)])
        pltpu.sync_copy(sums_loc, out_hbm.at[l, w])
        plsc.subcore_barrier()
        if l < 2:
            pltpu.sync_copy(v_sp, v_loc)


@functools.partial(
    pl.kernel,
    out_type=jax.ShapeDtypeStruct((3, NT, 16), jnp.float32),
    mesh=plsc.VectorSubcoreMesh(
        core_axis_name="c", subcore_axis_name="s",
        num_cores=1, num_subcores=NT),
    scratch_types=[
        pltpu.VMEM_SHARED((NT, NT, NPT), jnp.float32),
        pltpu.VMEM_SHARED((NP,), jnp.float32),
        pltpu.VMEM((EPT,), jnp.int32),
        pltpu.VMEM((EPT,), jnp.int32),
        pltpu.VMEM((NP,), jnp.float32),
        pltpu.VMEM((NP,), jnp.float32),
        pltpu.VMEM((NT, NPT), jnp.float32),
        pltpu.VMEM((NPT,), jnp.float32),
        pltpu.VMEM((NPT,), jnp.float32),
        pltpu.VMEM((16,), jnp.float32),
        pltpu.VMEM((16,), jnp.float32),
        pltpu.SemaphoreType.DMA,
    ],
    compiler_params=pltpu.CompilerParams(needs_layout_passes=False),
)
def _sc_edges(m0_hbm, s0_hbm, ei_hbm, scal_hbm, out_hbm,
              partials_sp, v_sp, src_v, dst_v, v_loc, agg_loc, pw_in,
              hnew, s0_loc, scal_loc, sums_loc, dma_sem):
    _sc_body(m0_hbm, s0_hbm, ei_hbm, scal_hbm, out_hbm,
             partials_sp, v_sp, src_v, dst_v, v_loc, agg_loc, pw_in,
             hnew, s0_loc, scal_loc, sums_loc, dma_sem)


def kernel(x, edge_index, W_neigh0, b0, W_self0, W_neigh1, b1, W_self1,
           W_neigh2, b2, W_self2):
    m0, s0, colsum = pl.pallas_call(
        _tc_body,
        out_shape=[
            jax.ShapeDtypeStruct((NP, 1), jnp.float32),
            jax.ShapeDtypeStruct((NP, 1), jnp.float32),
            jax.ShapeDtypeStruct((D,), jnp.float32),
        ],
    )(x, W_neigh0, W_self0)

    scal = jnp.concatenate([
        b0, W_neigh1[0], b1, W_self1[0], W_neigh2[0], b2, W_self2[0],
        jnp.zeros((9,), jnp.float32),
    ])

    sums = _sc_edges(m0.reshape(NP), s0.reshape(NP),
                     edge_index.reshape(2 * E), scal)
    hsums = jnp.sum(sums, axis=(1, 2))
    return jnp.concatenate([colsum, hsums])


# speedup vs baseline: 91.2136x; 1.0180x over previous
import functools

import jax
import jax.numpy as jnp
from jax import lax
from jax.experimental import pallas as pl
from jax.experimental.pallas import tpu as pltpu
from jax.experimental.pallas import tpu_sc as plsc

N = 10000
D = 128
E = 320000
NP = 10240
NT = 16
NPT = NP // NT
EPT = E // NT
EUNROLL = 10


def _tc_body(x_ref, wn_ref, ws_ref, b0r, wn1r, b1r, ws1r, wn2r, b2r, ws2r,
             m_ref, s_ref, cs_ref, sc_ref):
    xb = x_ref[...]
    zt = jnp.zeros((NP - N, 1), jnp.float32)
    m_ref[pl.ds(0, N), :] = jnp.dot(xb, wn_ref[...],
                                    preferred_element_type=jnp.float32)
    m_ref[pl.ds(N, NP - N), :] = zt
    s_ref[pl.ds(0, N), :] = jnp.dot(xb, ws_ref[...],
                                    preferred_element_type=jnp.float32)
    s_ref[pl.ds(N, NP - N), :] = zt
    cs_ref[...] = jnp.sum(xb, axis=0)
    sc_ref[...] = jnp.concatenate([
        b0r[...], wn1r[...].reshape(1), b1r[...], ws1r[...].reshape(1),
        wn2r[...].reshape(1), b2r[...], ws2r[...].reshape(1),
        jnp.zeros((9,), jnp.float32),
    ])


def _sc_body(m0_hbm, s0_hbm, cs_hbm, scal_hbm, ei_hbm, out_hbm,
             partials_sp, v_sp, sums_sp, src_v, dst_v, v_loc, agg_loc,
             pw_in, hnew, s0_loc, scal_loc, sums_loc, cs_loc, fin_loc,
             idx_loc, val_loc, dma_sem):
    w = lax.axis_index("s")
    nbase = w * NPT

    cp_src = pltpu.async_copy(ei_hbm.at[pl.ds(w * EPT, EPT)], src_v, dma_sem)
    cp_dst = pltpu.async_copy(ei_hbm.at[pl.ds(E + w * EPT, EPT)], dst_v,
                              dma_sem)
    pltpu.sync_copy(scal_hbm, scal_loc)
    pltpu.sync_copy(s0_hbm.at[pl.ds(nbase, NPT)], s0_loc)
    pltpu.sync_copy(m0_hbm, v_loc)
    cp_src.wait()
    cp_dst.wait()

    zero16 = jnp.zeros((16,), jnp.float32)
    iota16 = lax.iota(jnp.int32, 16)

    for l in range(3):
        @plsc.parallel_loop(0, NP // 128, unroll=4)
        def _(i):
            base = i * 128
            for u in range(8):
                agg_loc[pl.ds(base + u * 16, 16)] = zero16

        @plsc.parallel_loop(0, EPT // 16, unroll=EUNROLL)
        def _(i):
            off = i * 16
            sidx = src_v[pl.ds(off, 16)]
            didx = dst_v[pl.ds(off, 16)]
            vals = plsc.load_gather(v_loc, [sidx])
            plsc.addupdate_scatter(agg_loc, [didx], vals)

        cps = [pltpu.async_copy(agg_loc.at[pl.ds(t * NPT, NPT)],
                                partials_sp.at[t, w], dma_sem)
               for t in range(NT)]
        for cp in cps:
            cp.wait()
        plsc.subcore_barrier()
        pltpu.sync_copy(partials_sp.at[w], pw_in)

        scv = scal_loc[...]
        if l == 0:
            wb = scv[0]
        elif l == 1:
            wn, wb, ws = scv[1], scv[2], scv[3]
        else:
            wn, wb, ws = scv[4], scv[5], scv[6]

        @plsc.parallel_loop(0, NPT // 16, unroll=4, carry=zero16)
        def sacc(j, sc):
            off = j * 16
            acc = pw_in[0, pl.ds(off, 16)]
            for t in range(1, NT):
                acc = acc + pw_in[t, pl.ds(off, 16)]
            if l == 0:
                h = acc + wb + s0_loc[pl.ds(off, 16)]
            else:
                h = wn * acc + wb + ws * v_loc[pl.ds(nbase + off, 16)]
            h = jnp.maximum(h, 0.0)
            nid = nbase + off + iota16
            h = jnp.where(nid < N, h, 0.0)
            hnew[pl.ds(off, 16)] = h
            return sc + h
        sums_loc[...] = sacc
        pltpu.sync_copy(hnew, v_sp.at[pl.ds(nbase, NPT)])
        pltpu.sync_copy(sums_loc, sums_sp.at[pl.ds((l * NT + w) * 16, 16)])
        plsc.subcore_barrier()
        if l < 2:
            pltpu.sync_copy(v_sp, v_loc)

    @pl.when(w == 0)
    def _():
        pltpu.sync_copy(cs_hbm, cs_loc)
        pltpu.sync_copy(cs_loc, out_hbm.at[pl.ds(0, D)])
        pltpu.sync_copy(sums_sp, fin_loc)
        tots = []
        for l in range(3):
            acc = fin_loc[pl.ds(l * NT * 16, 16)]
            for t in range(1, NT):
                acc = acc + fin_loc[pl.ds((l * NT + t) * 16, 16)]
            tots.append(jnp.sum(acc))
        val = jnp.where(iota16 == 0, tots[0],
                        jnp.where(iota16 == 1, tots[1],
                                  jnp.where(iota16 == 2, tots[2], 0.0)))
        val_loc[...] = val
        pltpu.sync_copy(val_loc, out_hbm.at[pl.ds(D, 16)])


@functools.partial(
    pl.kernel,
    out_type=jax.ShapeDtypeStruct((D + 16,), jnp.float32),
    mesh=plsc.VectorSubcoreMesh(
        core_axis_name="c", subcore_axis_name="s",
        num_cores=1, num_subcores=NT),
    scratch_types=[
        pltpu.VMEM_SHARED((NT, NT, NPT), jnp.float32),
        pltpu.VMEM_SHARED((NP,), jnp.float32),
        pltpu.VMEM_SHARED((3 * NT * 16,), jnp.float32),
        pltpu.VMEM((EPT,), jnp.int32),
        pltpu.VMEM((EPT,), jnp.int32),
        pltpu.VMEM((NP,), jnp.float32),
        pltpu.VMEM((NP,), jnp.float32),
        pltpu.VMEM((NT, NPT), jnp.float32),
        pltpu.VMEM((NPT,), jnp.float32),
        pltpu.VMEM((NPT,), jnp.float32),
        pltpu.VMEM((16,), jnp.float32),
        pltpu.VMEM((16,), jnp.float32),
        pltpu.VMEM((D,), jnp.float32),
        pltpu.VMEM((3 * NT * 16,), jnp.float32),
        pltpu.VMEM((16,), jnp.int32),
        pltpu.VMEM((16,), jnp.float32),
        pltpu.SemaphoreType.DMA,
    ],
    compiler_params=pltpu.CompilerParams(needs_layout_passes=False),
)
def _sc_edges(m0_hbm, s0_hbm, cs_hbm, scal_hbm, ei_hbm, out_hbm,
              partials_sp, v_sp, sums_sp, src_v, dst_v, v_loc, agg_loc,
              pw_in, hnew, s0_loc, scal_loc, sums_loc, cs_loc, fin_loc,
              idx_loc, val_loc, dma_sem):
    _sc_body(m0_hbm, s0_hbm, cs_hbm, scal_hbm, ei_hbm, out_hbm,
             partials_sp, v_sp, sums_sp, src_v, dst_v, v_loc, agg_loc,
             pw_in, hnew, s0_loc, scal_loc, sums_loc, cs_loc, fin_loc,
             idx_loc, val_loc, dma_sem)


def kernel(x, edge_index, W_neigh0, b0, W_self0, W_neigh1, b1, W_self1,
           W_neigh2, b2, W_self2):
    m0, s0, colsum, scal16 = pl.pallas_call(
        _tc_body,
        out_shape=[
            jax.ShapeDtypeStruct((NP, 1), jnp.float32),
            jax.ShapeDtypeStruct((NP, 1), jnp.float32),
            jax.ShapeDtypeStruct((D,), jnp.float32),
            jax.ShapeDtypeStruct((16,), jnp.float32),
        ],
    )(x, W_neigh0, W_self0, b0, W_neigh1, b1, W_self1, W_neigh2, b2, W_self2)

    out = _sc_edges(m0.reshape(NP), s0.reshape(NP), colsum, scal16,
                    edge_index.reshape(2 * E))
    return out[:D + 3]


# speedup vs baseline: 91.3073x; 1.0010x over previous
import functools

import jax
import jax.numpy as jnp
from jax import lax
from jax.experimental import pallas as pl
from jax.experimental.pallas import tpu as pltpu
from jax.experimental.pallas import tpu_sc as plsc

N = 10000
D = 128
E = 320000
NP = 10240
NT = 16
NPT = NP // NT
EPT = E // NT
EUNROLL = 25


def _tc_body(x_ref, wn_ref, ws_ref, b0r, wn1r, b1r, ws1r, wn2r, b2r, ws2r,
             m_ref, s_ref, cs_ref, sc_ref):
    xb = x_ref[...]
    zt = jnp.zeros((NP - N, 1), jnp.float32)
    m_ref[pl.ds(0, N), :] = jnp.dot(xb, wn_ref[...],
                                    preferred_element_type=jnp.float32)
    m_ref[pl.ds(N, NP - N), :] = zt
    s_ref[pl.ds(0, N), :] = jnp.dot(xb, ws_ref[...],
                                    preferred_element_type=jnp.float32)
    s_ref[pl.ds(N, NP - N), :] = zt
    cs_ref[...] = jnp.sum(xb, axis=0)
    sc_ref[...] = jnp.concatenate([
        b0r[...], wn1r[...].reshape(1), b1r[...], ws1r[...].reshape(1),
        wn2r[...].reshape(1), b2r[...], ws2r[...].reshape(1),
        jnp.zeros((9,), jnp.float32),
    ])


def _sc_body(m0_hbm, s0_hbm, cs_hbm, scal_hbm, ei_hbm, out_hbm,
             partials_sp, v_sp, sums_sp, src_v, dst_v, v_loc, agg_loc,
             pw_in, hnew, s0_loc, scal_loc, sums_loc, cs_loc, fin_loc,
             idx_loc, val_loc, dma_sem):
    w = lax.axis_index("s")
    nbase = w * NPT

    cp_src = pltpu.async_copy(ei_hbm.at[pl.ds(w * EPT, EPT)], src_v, dma_sem)
    cp_dst = pltpu.async_copy(ei_hbm.at[pl.ds(E + w * EPT, EPT)], dst_v,
                              dma_sem)
    pltpu.sync_copy(scal_hbm, scal_loc)
    pltpu.sync_copy(s0_hbm.at[pl.ds(nbase, NPT)], s0_loc)
    pltpu.sync_copy(m0_hbm, v_loc)
    cp_src.wait()
    cp_dst.wait()

    zero16 = jnp.zeros((16,), jnp.float32)
    iota16 = lax.iota(jnp.int32, 16)

    for l in range(3):
        @plsc.parallel_loop(0, NP // 128, unroll=4)
        def _(i):
            base = i * 128
            for u in range(8):
                agg_loc[pl.ds(base + u * 16, 16)] = zero16

        @plsc.parallel_loop(0, EPT // 16, unroll=EUNROLL)
        def _(i):
            off = i * 16
            sidx = src_v[pl.ds(off, 16)]
            didx = dst_v[pl.ds(off, 16)]
            vals = plsc.load_gather(v_loc, [sidx])
            plsc.addupdate_scatter(agg_loc, [didx], vals)

        cps = [pltpu.async_copy(agg_loc.at[pl.ds(t * NPT, NPT)],
                                partials_sp.at[t, w], dma_sem)
               for t in range(NT)]
        for cp in cps:
            cp.wait()
        plsc.subcore_barrier()
        pltpu.sync_copy(partials_sp.at[w], pw_in)

        scv = scal_loc[...]
        if l == 0:
            wb = scv[0]
        elif l == 1:
            wn, wb, ws = scv[1], scv[2], scv[3]
        else:
            wn, wb, ws = scv[4], scv[5], scv[6]

        @plsc.parallel_loop(0, NPT // 16, unroll=4, carry=zero16)
        def sacc(j, sc):
            off = j * 16
            acc = pw_in[0, pl.ds(off, 16)]
            for t in range(1, NT):
                acc = acc + pw_in[t, pl.ds(off, 16)]
            if l == 0:
                h = acc + wb + s0_loc[pl.ds(off, 16)]
            else:
                h = wn * acc + wb + ws * v_loc[pl.ds(nbase + off, 16)]
            h = jnp.maximum(h, 0.0)
            nid = nbase + off + iota16
            h = jnp.where(nid < N, h, 0.0)
            hnew[pl.ds(off, 16)] = h
            return sc + h
        sums_loc[...] = sacc
        pltpu.sync_copy(hnew, v_sp.at[pl.ds(nbase, NPT)])
        pltpu.sync_copy(sums_loc, sums_sp.at[pl.ds((l * NT + w) * 16, 16)])
        plsc.subcore_barrier()
        if l < 2:
            pltpu.sync_copy(v_sp, v_loc)

    @pl.when(w == 0)
    def _():
        pltpu.sync_copy(cs_hbm, cs_loc)
        pltpu.sync_copy(cs_loc, out_hbm.at[pl.ds(0, D)])
        pltpu.sync_copy(sums_sp, fin_loc)
        tots = []
        for l in range(3):
            acc = fin_loc[pl.ds(l * NT * 16, 16)]
            for t in range(1, NT):
                acc = acc + fin_loc[pl.ds((l * NT + t) * 16, 16)]
            tots.append(jnp.sum(acc))
        val = jnp.where(iota16 == 0, tots[0],
                        jnp.where(iota16 == 1, tots[1],
                                  jnp.where(iota16 == 2, tots[2], 0.0)))
        val_loc[...] = val
        pltpu.sync_copy(val_loc, out_hbm.at[pl.ds(D, 16)])


@functools.partial(
    pl.kernel,
    out_type=jax.ShapeDtypeStruct((D + 16,), jnp.float32),
    mesh=plsc.VectorSubcoreMesh(
        core_axis_name="c", subcore_axis_name="s",
        num_cores=1, num_subcores=NT),
    scratch_types=[
        pltpu.VMEM_SHARED((NT, NT, NPT), jnp.float32),
        pltpu.VMEM_SHARED((NP,), jnp.float32),
        pltpu.VMEM_SHARED((3 * NT * 16,), jnp.float32),
        pltpu.VMEM((EPT,), jnp.int32),
        pltpu.VMEM((EPT,), jnp.int32),
        pltpu.VMEM((NP,), jnp.float32),
        pltpu.VMEM((NP,), jnp.float32),
        pltpu.VMEM((NT, NPT), jnp.float32),
        pltpu.VMEM((NPT,), jnp.float32),
        pltpu.VMEM((NPT,), jnp.float32),
        pltpu.VMEM((16,), jnp.float32),
        pltpu.VMEM((16,), jnp.float32),
        pltpu.VMEM((D,), jnp.float32),
        pltpu.VMEM((3 * NT * 16,), jnp.float32),
        pltpu.VMEM((16,), jnp.int32),
        pltpu.VMEM((16,), jnp.float32),
        pltpu.SemaphoreType.DMA,
    ],
    compiler_params=pltpu.CompilerParams(needs_layout_passes=False),
)
def _sc_edges(m0_hbm, s0_hbm, cs_hbm, scal_hbm, ei_hbm, out_hbm,
              partials_sp, v_sp, sums_sp, src_v, dst_v, v_loc, agg_loc,
              pw_in, hnew, s0_loc, scal_loc, sums_loc, cs_loc, fin_loc,
              idx_loc, val_loc, dma_sem):
    _sc_body(m0_hbm, s0_hbm, cs_hbm, scal_hbm, ei_hbm, out_hbm,
             partials_sp, v_sp, sums_sp, src_v, dst_v, v_loc, agg_loc,
             pw_in, hnew, s0_loc, scal_loc, sums_loc, cs_loc, fin_loc,
             idx_loc, val_loc, dma_sem)


def kernel(x, edge_index, W_neigh0, b0, W_self0, W_neigh1, b1, W_self1,
           W_neigh2, b2, W_self2):
    m0, s0, colsum, scal16 = pl.pallas_call(
        _tc_body,
        out_shape=[
            jax.ShapeDtypeStruct((NP, 1), jnp.float32),
            jax.ShapeDtypeStruct((NP, 1), jnp.float32),
            jax.ShapeDtypeStruct((D,), jnp.float32),
            jax.ShapeDtypeStruct((16,), jnp.float32),
        ],
    )(x, W_neigh0, W_self0, b0, W_neigh1, b1, W_self1, W_neigh2, b2, W_self2)

    out = _sc_edges(m0.reshape(NP), s0.reshape(NP), colsum, scal16,
                    edge_index.reshape(2 * E))
    return out[:D + 3]


# speedup vs baseline: 92.8509x; 1.0169x over previous
import functools

import jax
import jax.numpy as jnp
from jax import lax
from jax.experimental import pallas as pl
from jax.experimental.pallas import tpu as pltpu
from jax.experimental.pallas import tpu_sc as plsc

N = 10000
D = 128
E = 320000
NP = 10240
NT = 16
NPT = NP // NT
EPT = E // NT
EUNROLL = 25


def _tc_body(x_ref, wn_ref, ws_ref, b0r, wn1r, b1r, ws1r, wn2r, b2r, ws2r,
             m_ref, s_ref, cs_ref, sc_ref):
    xb = x_ref[...]
    zt = jnp.zeros((NP - N, 1), jnp.float32)
    m_ref[pl.ds(0, N), :] = jnp.dot(xb, wn_ref[...],
                                    preferred_element_type=jnp.float32)
    m_ref[pl.ds(N, NP - N), :] = zt
    s_ref[pl.ds(0, N), :] = jnp.dot(xb, ws_ref[...],
                                    preferred_element_type=jnp.float32)
    s_ref[pl.ds(N, NP - N), :] = zt
    cs_ref[...] = jnp.sum(xb, axis=0)
    sc_ref[...] = jnp.concatenate([
        b0r[...], wn1r[...].reshape(1), b1r[...], ws1r[...].reshape(1),
        wn2r[...].reshape(1), b2r[...], ws2r[...].reshape(1),
        jnp.zeros((9,), jnp.float32),
    ])


def _sc_body(m0_hbm, s0_hbm, cs_hbm, scal_hbm, ei_hbm, out_hbm,
             partials_sp, v_sp, sums_sp, src_v, dst_v, v_loc, agg_loc,
             pw_in, hnew, s0_loc, scal_loc, sums_loc, cs_loc, fin_loc,
             idx_loc, val_loc, dma_sem):
    w = lax.axis_index("s")
    nbase = w * NPT

    cps0 = [
        pltpu.async_copy(ei_hbm.at[pl.ds(w * EPT, EPT)], src_v, dma_sem),
        pltpu.async_copy(ei_hbm.at[pl.ds(E + w * EPT, EPT)], dst_v, dma_sem),
        pltpu.async_copy(scal_hbm, scal_loc, dma_sem),
        pltpu.async_copy(s0_hbm.at[pl.ds(nbase, NPT)], s0_loc, dma_sem),
        pltpu.async_copy(m0_hbm, v_loc, dma_sem),
    ]

    zero16 = jnp.zeros((16,), jnp.float32)
    iota16 = lax.iota(jnp.int32, 16)

    for l in range(3):
        @plsc.parallel_loop(0, NP // 128, unroll=4)
        def _(i):
            base = i * 128
            for u in range(8):
                agg_loc[pl.ds(base + u * 16, 16)] = zero16

        if l == 0:
            for cp in cps0:
                cp.wait()
        else:
            cp_v.wait()

        @plsc.parallel_loop(0, EPT // 16, unroll=EUNROLL)
        def _(i):
            off = i * 16
            sidx = src_v[pl.ds(off, 16)]
            didx = dst_v[pl.ds(off, 16)]
            vals = plsc.load_gather(v_loc, [sidx])
            plsc.addupdate_scatter(agg_loc, [didx], vals)

        cps = [pltpu.async_copy(agg_loc.at[pl.ds(t * NPT, NPT)],
                                partials_sp.at[t, w], dma_sem)
               for t in range(NT)]
        for cp in cps:
            cp.wait()
        plsc.subcore_barrier()
        pltpu.sync_copy(partials_sp.at[w], pw_in)

        scv = scal_loc[...]
        if l == 0:
            wb = scv[0]
        elif l == 1:
            wn, wb, ws = scv[1], scv[2], scv[3]
        else:
            wn, wb, ws = scv[4], scv[5], scv[6]

        @plsc.parallel_loop(0, NPT // 16, unroll=4, carry=zero16)
        def sacc(j, sc):
            off = j * 16
            acc = pw_in[0, pl.ds(off, 16)]
            for t in range(1, NT):
                acc = acc + pw_in[t, pl.ds(off, 16)]
            if l == 0:
                h = acc + wb + s0_loc[pl.ds(off, 16)]
            else:
                h = wn * acc + wb + ws * v_loc[pl.ds(nbase + off, 16)]
            h = jnp.maximum(h, 0.0)
            nid = nbase + off + iota16
            h = jnp.where(nid < N, h, 0.0)
            hnew[pl.ds(off, 16)] = h
            return sc + h
        sums_loc[...] = sacc
        pltpu.sync_copy(hnew, v_sp.at[pl.ds(nbase, NPT)])
        pltpu.sync_copy(sums_loc, sums_sp.at[pl.ds((l * NT + w) * 16, 16)])
        plsc.subcore_barrier()
        if l < 2:
            cp_v = pltpu.async_copy(v_sp, v_loc, dma_sem)

    @pl.when(w == 0)
    def _():
        pltpu.sync_copy(cs_hbm, cs_loc)
        pltpu.sync_copy(cs_loc, out_hbm.at[pl.ds(0, D)])
        pltpu.sync_copy(sums_sp, fin_loc)
        tots = []
        for l in range(3):
            acc = fin_loc[pl.ds(l * NT * 16, 16)]
            for t in range(1, NT):
                acc = acc + fin_loc[pl.ds((l * NT + t) * 16, 16)]
            tots.append(jnp.sum(acc))
        val = jnp.where(iota16 == 0, tots[0],
                        jnp.where(iota16 == 1, tots[1],
                                  jnp.where(iota16 == 2, tots[2], 0.0)))
        val_loc[...] = val
        pltpu.sync_copy(val_loc, out_hbm.at[pl.ds(D, 16)])


@functools.partial(
    pl.kernel,
    out_type=jax.ShapeDtypeStruct((D + 16,), jnp.float32),
    mesh=plsc.VectorSubcoreMesh(
        core_axis_name="c", subcore_axis_name="s",
        num_cores=1, num_subcores=NT),
    scratch_types=[
        pltpu.VMEM_SHARED((NT, NT, NPT), jnp.float32),
        pltpu.VMEM_SHARED((NP,), jnp.float32),
        pltpu.VMEM_SHARED((3 * NT * 16,), jnp.float32),
        pltpu.VMEM((EPT,), jnp.int32),
        pltpu.VMEM((EPT,), jnp.int32),
        pltpu.VMEM((NP,), jnp.float32),
        pltpu.VMEM((NP,), jnp.float32),
        pltpu.VMEM((NT, NPT), jnp.float32),
        pltpu.VMEM((NPT,), jnp.float32),
        pltpu.VMEM((NPT,), jnp.float32),
        pltpu.VMEM((16,), jnp.float32),
        pltpu.VMEM((16,), jnp.float32),
        pltpu.VMEM((D,), jnp.float32),
        pltpu.VMEM((3 * NT * 16,), jnp.float32),
        pltpu.VMEM((16,), jnp.int32),
        pltpu.VMEM((16,), jnp.float32),
        pltpu.SemaphoreType.DMA,
    ],
    compiler_params=pltpu.CompilerParams(needs_layout_passes=False),
)
def _sc_edges(m0_hbm, s0_hbm, cs_hbm, scal_hbm, ei_hbm, out_hbm,
              partials_sp, v_sp, sums_sp, src_v, dst_v, v_loc, agg_loc,
              pw_in, hnew, s0_loc, scal_loc, sums_loc, cs_loc, fin_loc,
              idx_loc, val_loc, dma_sem):
    _sc_body(m0_hbm, s0_hbm, cs_hbm, scal_hbm, ei_hbm, out_hbm,
             partials_sp, v_sp, sums_sp, src_v, dst_v, v_loc, agg_loc,
             pw_in, hnew, s0_loc, scal_loc, sums_loc, cs_loc, fin_loc,
             idx_loc, val_loc, dma_sem)


def kernel(x, edge_index, W_neigh0, b0, W_self0, W_neigh1, b1, W_self1,
           W_neigh2, b2, W_self2):
    m0, s0, colsum, scal16 = pl.pallas_call(
        _tc_body,
        out_shape=[
            jax.ShapeDtypeStruct((NP, 1), jnp.float32),
            jax.ShapeDtypeStruct((NP, 1), jnp.float32),
            jax.ShapeDtypeStruct((D,), jnp.float32),
            jax.ShapeDtypeStruct((16,), jnp.float32),
        ],
    )(x, W_neigh0, W_self0, b0, W_neigh1, b1, W_self1, W_neigh2, b2, W_self2)

    out = _sc_edges(m0.reshape(NP), s0.reshape(NP), colsum, scal16,
                    edge_index.reshape(2 * E))
    return out[:D + 3]


# speedup vs baseline: 92.9144x; 1.0007x over previous
import functools

import jax
import jax.numpy as jnp
from jax import lax
from jax.experimental import pallas as pl
from jax.experimental.pallas import tpu as pltpu
from jax.experimental.pallas import tpu_sc as plsc

N = 10000
D = 128
E = 320000
NP = 10240
NT = 16
NPT = NP // NT
EPT = E // NT
EUNROLL = 25


def _tc_body(x_ref, wn_ref, ws_ref, b0r, wn1r, b1r, ws1r, wn2r, b2r, ws2r,
             m_ref, s_ref, cs_ref, sc_ref):
    xb = x_ref[...]
    zt = jnp.zeros((NP - N, 1), jnp.float32)
    m_ref[pl.ds(0, N), :] = jnp.dot(xb, wn_ref[...],
                                    preferred_element_type=jnp.float32)
    m_ref[pl.ds(N, NP - N), :] = zt
    s_ref[pl.ds(0, N), :] = jnp.dot(xb, ws_ref[...],
                                    preferred_element_type=jnp.float32)
    s_ref[pl.ds(N, NP - N), :] = zt
    cs_ref[...] = jnp.sum(xb, axis=0)
    sc_ref[...] = jnp.concatenate([
        b0r[...], wn1r[...].reshape(1), b1r[...], ws1r[...].reshape(1),
        wn2r[...].reshape(1), b2r[...], ws2r[...].reshape(1),
        jnp.zeros((9,), jnp.float32),
    ])


def _sc_body(m0_hbm, s0_hbm, cs_hbm, scal_hbm, ei_hbm, out_hbm,
             partials_sp, v_sp, sums_sp, src_v, dst_v, v_loc, agg_loc,
             pw_in, hnew, s0_loc, scal_loc, sums_loc, cs_loc, fin_loc,
             val_loc, dma_sem):
    w = lax.axis_index("s")
    nbase = w * NPT

    cps0 = [
        pltpu.async_copy(ei_hbm.at[pl.ds(w * EPT, EPT)], src_v, dma_sem),
        pltpu.async_copy(ei_hbm.at[pl.ds(E + w * EPT, EPT)], dst_v, dma_sem),
        pltpu.async_copy(scal_hbm, scal_loc, dma_sem),
        pltpu.async_copy(s0_hbm.at[pl.ds(nbase, NPT)], s0_loc, dma_sem),
        pltpu.async_copy(m0_hbm, v_loc, dma_sem),
    ]

    zero16 = jnp.zeros((16,), jnp.float32)
    iota16 = lax.iota(jnp.int32, 16)

    for l in range(3):
        @plsc.parallel_loop(0, NP // 128, unroll=4)
        def _(i):
            base = i * 128
            for u in range(8):
                agg_loc[pl.ds(base + u * 16, 16)] = zero16

        if l == 0:
            for cp in cps0:
                cp.wait()
        else:
            cp_v.wait()

        @plsc.parallel_loop(0, EPT // 16, unroll=EUNROLL)
        def _(i):
            off = i * 16
            sidx = src_v[pl.ds(off, 16)]
            didx = dst_v[pl.ds(off, 16)]
            vals = plsc.load_gather(v_loc, [sidx])
            plsc.addupdate_scatter(agg_loc, [didx], vals)

        cps = [pltpu.async_copy(agg_loc.at[pl.ds(t * NPT, NPT)],
                                partials_sp.at[t, w], dma_sem)
               for t in range(NT)]
        for cp in cps:
            cp.wait()
        plsc.subcore_barrier()
        pltpu.sync_copy(partials_sp.at[w], pw_in)

        scv = scal_loc[...]
        if l == 0:
            wb = scv[0]
        elif l == 1:
            wn, wb, ws = scv[1], scv[2], scv[3]
        else:
            wn, wb, ws = scv[4], scv[5], scv[6]

        @plsc.parallel_loop(0, NPT // 16, unroll=4, carry=zero16)
        def sacc(j, sc):
            off = j * 16
            acc = pw_in[0, pl.ds(off, 16)]
            for t in range(1, NT):
                acc = acc + pw_in[t, pl.ds(off, 16)]
            if l == 0:
                h = acc + wb + s0_loc[pl.ds(off, 16)]
            else:
                h = wn * acc + wb + ws * v_loc[pl.ds(nbase + off, 16)]
            h = jnp.maximum(h, 0.0)
            nid = nbase + off + iota16
            h = jnp.where(nid < N, h, 0.0)
            hnew[pl.ds(off, 16)] = h
            return sc + h
        sums_loc[...] = sacc
        pltpu.sync_copy(hnew, v_sp.at[pl.ds(nbase, NPT)])
        pltpu.sync_copy(sums_loc, sums_sp.at[pl.ds((l * NT + w) * 16, 16)])
        plsc.subcore_barrier()
        if l < 2:
            cp_v = pltpu.async_copy(v_sp, v_loc, dma_sem)

    @pl.when(w == 0)
    def _():
        pltpu.sync_copy(cs_hbm, cs_loc)
        pltpu.sync_copy(cs_loc, out_hbm.at[pl.ds(0, D)])
        pltpu.sync_copy(sums_sp, fin_loc)
        tots = []
        for l in range(3):
            acc = fin_loc[pl.ds(l * NT * 16, 16)]
            for t in range(1, NT):
                acc = acc + fin_loc[pl.ds((l * NT + t) * 16, 16)]
            tots.append(jnp.sum(acc))
        val = jnp.where(iota16 == 0, tots[0],
                        jnp.where(iota16 == 1, tots[1],
                                  jnp.where(iota16 == 2, tots[2], 0.0)))
        val_loc[...] = val
        pltpu.sync_copy(val_loc, out_hbm.at[pl.ds(D, 16)])


@functools.partial(
    pl.kernel,
    out_type=jax.ShapeDtypeStruct((D + 16,), jnp.float32),
    mesh=plsc.VectorSubcoreMesh(
        core_axis_name="c", subcore_axis_name="s",
        num_cores=1, num_subcores=NT),
    scratch_types=[
        pltpu.VMEM_SHARED((NT, NT, NPT), jnp.float32),
        pltpu.VMEM_SHARED((NP,), jnp.float32),
        pltpu.VMEM_SHARED((3 * NT * 16,), jnp.float32),
        pltpu.VMEM((EPT,), jnp.int32),
        pltpu.VMEM((EPT,), jnp.int32),
        pltpu.VMEM((NP,), jnp.float32),
        pltpu.VMEM((NP,), jnp.float32),
        pltpu.VMEM((NT, NPT), jnp.float32),
        pltpu.VMEM((NPT,), jnp.float32),
        pltpu.VMEM((NPT,), jnp.float32),
        pltpu.VMEM((16,), jnp.float32),
        pltpu.VMEM((16,), jnp.float32),
        pltpu.VMEM((D,), jnp.float32),
        pltpu.VMEM((3 * NT * 16,), jnp.float32),
        pltpu.VMEM((16,), jnp.float32),
        pltpu.SemaphoreType.DMA,
    ],
    compiler_params=pltpu.CompilerParams(needs_layout_passes=False),
)
def _sc_edges(m0_hbm, s0_hbm, cs_hbm, scal_hbm, ei_hbm, out_hbm,
              partials_sp, v_sp, sums_sp, src_v, dst_v, v_loc, agg_loc,
              pw_in, hnew, s0_loc, scal_loc, sums_loc, cs_loc, fin_loc,
              val_loc, dma_sem):
    _sc_body(m0_hbm, s0_hbm, cs_hbm, scal_hbm, ei_hbm, out_hbm,
             partials_sp, v_sp, sums_sp, src_v, dst_v, v_loc, agg_loc,
             pw_in, hnew, s0_loc, scal_loc, sums_loc, cs_loc, fin_loc,
             val_loc, dma_sem)


def kernel(x, edge_index, W_neigh0, b0, W_self0, W_neigh1, b1, W_self1,
           W_neigh2, b2, W_self2):
    m0, s0, colsum, scal16 = pl.pallas_call(
        _tc_body,
        out_shape=[
            jax.ShapeDtypeStruct((NP, 1), jnp.float32),
            jax.ShapeDtypeStruct((NP, 1), jnp.float32),
            jax.ShapeDtypeStruct((D,), jnp.float32),
            jax.ShapeDtypeStruct((16,), jnp.float32),
        ],
    )(x, W_neigh0, W_self0, b0, W_neigh1, b1, W_self1, W_neigh2, b2, W_self2)

    out = _sc_edges(m0.reshape(NP), s0.reshape(NP), colsum, scal16,
                    edge_index.reshape(2 * E))
    return out[:D + 3]
